# baseline (speedup 1.0000x reference)
import jax
import jax.numpy as jnp
from jax.experimental import pallas as pl


def _mlp_kernel(emb_ref, w1_ref, b1_ref, out_ref):
    x = (emb_ref[0:1, :] + emb_ref[1:2, :]) * 0.5
    y = jax.lax.dot_general(
        x, w1_ref[...], (((1,), (1,)), ((), ())),
        preferred_element_type=jnp.float32)
    out_ref[...] = jnp.maximum(y + b1_ref[...], 0.0)


def kernel(label, embedding, W1, b1):
    del label
    emb_dim = embedding.shape[1]
    hid = W1.shape[0]
    return pl.pallas_call(
        _mlp_kernel,
        grid=(1,),
        out_shape=jax.ShapeDtypeStruct((1, hid), jnp.float32),
        in_specs=[
            pl.BlockSpec((8, emb_dim), lambda i: (0, 0)),
            pl.BlockSpec(W1.shape, lambda i: (0, 0)),
            pl.BlockSpec((1, hid), lambda i: (0, 0)),
        ],
        out_specs=pl.BlockSpec((1, hid), lambda i: (0, 0)),
    )(embedding, W1, b1.reshape(1, hid))
